# baseline (speedup 1.0000x reference)
import functools

import jax
import jax.numpy as jnp
from jax import lax
from jax.experimental import pallas as pl
from jax.experimental.pallas import tpu as pltpu
from jax.experimental.pallas import tpu_sc as plsc

D_MODEL = 2048
N_ROWS = 4 * 4096
NUM_WORKERS = 32
ROWS_PER_WORKER = N_ROWS // NUM_WORKERS
CHUNK = 32
NUM_CHUNKS = ROWS_PER_WORKER // CHUNK

_mesh = plsc.VectorSubcoreMesh(core_axis_name="c", subcore_axis_name="s")


@functools.partial(
    pl.kernel,
    mesh=_mesh,
    out_type=jax.ShapeDtypeStruct((N_ROWS, D_MODEL), jnp.float32),
    scratch_types=[
        pltpu.VMEM((ROWS_PER_WORKER,), jnp.int32),
        pltpu.VMEM((CHUNK, D_MODEL), jnp.float32),
        pltpu.SemaphoreType.DMA,
    ],
)
def _embed_sc(idx_hbm, table_hbm, out_hbm, idx_v, rows_v, sem):
    wid = lax.axis_index("s") * 2 + lax.axis_index("c")
    base = wid * ROWS_PER_WORKER
    pltpu.sync_copy(idx_hbm.at[pl.ds(base, ROWS_PER_WORKER)], idx_v)
    for c in range(NUM_CHUNKS):
        off = c * CHUNK
        pltpu.async_copy(
            table_hbm.at[idx_v.at[pl.ds(off, CHUNK)]], rows_v, sem
        ).wait()
        pltpu.sync_copy(rows_v, out_hbm.at[pl.ds(base + off, CHUNK)])


def kernel(x, emb_weight):
    idx = x.reshape(-1).astype(jnp.int32)
    out = _embed_sc(idx, emb_weight)
    return out.reshape(x.shape + (D_MODEL,))


# speedup vs baseline: 8.3619x; 8.3619x over previous
import functools

import jax
import jax.numpy as jnp
from jax import lax
from jax.experimental import pallas as pl
from jax.experimental.pallas import tpu as pltpu
from jax.experimental.pallas import tpu_sc as plsc

D_MODEL = 2048
N_ROWS = 4 * 4096
NUM_WORKERS = 32
ROWS_PER_WORKER = N_ROWS // NUM_WORKERS

_mesh = plsc.VectorSubcoreMesh(core_axis_name="c", subcore_axis_name="s")


@functools.partial(
    pl.kernel,
    mesh=_mesh,
    out_type=jax.ShapeDtypeStruct((N_ROWS, D_MODEL), jnp.float32),
    scratch_types=[
        pltpu.VMEM((2, D_MODEL), jnp.float32),
        pltpu.VMEM((ROWS_PER_WORKER,), jnp.int32),
        pltpu.SemaphoreType.DMA,
    ],
)
def _embed_sc(idx_hbm, table_hbm, out_hbm, table_v, idx_v, sem):
    wid = lax.axis_index("s") * 2 + lax.axis_index("c")
    base = wid * ROWS_PER_WORKER
    pltpu.sync_copy(table_hbm, table_v)
    pltpu.sync_copy(idx_hbm.at[pl.ds(base, ROWS_PER_WORKER)], idx_v)
    @pl.loop(0, ROWS_PER_WORKER // 16)
    def _groups(g):
        off = g * 16
        xv = idx_v[pl.ds(off, 16)]
        for l in range(16):
            pltpu.async_copy(
                table_v.at[xv[l]], out_hbm.at[base + off + l], sem
            )

    slab = out_hbm.at[pl.ds(base, ROWS_PER_WORKER)]
    pltpu.make_async_copy(slab, slab, sem).wait()


def kernel(x, emb_weight):
    idx = x.reshape(-1).astype(jnp.int32)
    out = _embed_sc(idx, emb_weight)
    return out.reshape(x.shape + (D_MODEL,))


# speedup vs baseline: 8.3631x; 1.0001x over previous
import functools

import jax
import jax.numpy as jnp
from jax import lax
from jax.experimental import pallas as pl
from jax.experimental.pallas import tpu as pltpu
from jax.experimental.pallas import tpu_sc as plsc

BATCH = 4
SEQ = 4096
D_MODEL = 2048
NUM_WORKERS = 32
ROWS_PER_WORKER = BATCH * SEQ // NUM_WORKERS
SLABS_PER_BATCH = SEQ // ROWS_PER_WORKER

_mesh = plsc.VectorSubcoreMesh(core_axis_name="c", subcore_axis_name="s")


@functools.partial(
    pl.kernel,
    mesh=_mesh,
    out_type=jax.ShapeDtypeStruct((BATCH, SEQ, D_MODEL), jnp.float32),
    scratch_types=[
        pltpu.VMEM((2, D_MODEL), jnp.float32),
        pltpu.VMEM((ROWS_PER_WORKER,), jnp.int32),
        pltpu.SemaphoreType.DMA,
    ],
)
def _embed_sc(x_hbm, table_hbm, out_hbm, table_v, idx_v, sem):
    wid = lax.axis_index("s") * 2 + lax.axis_index("c")
    b = wid // SLABS_PER_BATCH
    s0 = (wid % SLABS_PER_BATCH) * ROWS_PER_WORKER
    pltpu.sync_copy(table_hbm, table_v)
    pltpu.sync_copy(x_hbm.at[b, pl.ds(s0, ROWS_PER_WORKER)], idx_v)

    @pl.loop(0, ROWS_PER_WORKER // 16)
    def _groups(g):
        off = g * 16
        xv = idx_v[pl.ds(off, 16)]
        for l in range(16):
            pltpu.async_copy(
                table_v.at[xv[l]], out_hbm.at[b, s0 + off + l], sem
            )

    slab = out_hbm.at[b, pl.ds(s0, ROWS_PER_WORKER)]
    pltpu.make_async_copy(slab, slab, sem).wait()


def kernel(x, emb_weight):
    return _embed_sc(x.astype(jnp.int32), emb_weight)
